# Initial kernel scaffold; baseline (speedup 1.0000x reference)
import jax
import jax.numpy as jnp
from jax.experimental import pallas as pl


def kernel(x, edge_index, edge_weight, W1, b1, W2, b2):
    raise NotImplementedError("write your pallas kernel here")


# speedup vs baseline: 3.5254x; 3.5254x over previous
import functools

import jax
import jax.numpy as jnp
from jax import lax
from jax.experimental import pallas as pl
from jax.experimental.pallas import tpu as pltpu
from jax.experimental.pallas import tpu_sc as plsc

_NC = 2
_NS = 16
_K = 40


def _mm2_body(x_ref, wa_ref, wb_ref, oa_ref, ob_ref):
    x = x_ref[...]
    oa_ref[...] = jnp.dot(x, wa_ref[...], preferred_element_type=jnp.float32)
    ob_ref[...] = jnp.dot(x, wb_ref[...], preferred_element_type=jnp.float32)


def _matmul_split(x, wa, wb, blk):
    n, d = x.shape
    ha, hb = wa.shape[1], wb.shape[1]
    return pl.pallas_call(
        _mm2_body,
        grid=(n // blk,),
        in_specs=[pl.BlockSpec((blk, d), lambda i: (i, 0)),
                  pl.BlockSpec((d, ha), lambda i: (0, 0)),
                  pl.BlockSpec((d, hb), lambda i: (0, 0))],
        out_specs=[pl.BlockSpec((blk, ha), lambda i: (i, 0)),
                   pl.BlockSpec((blk, hb), lambda i: (i, 0))],
        out_shape=[jax.ShapeDtypeStruct((n, ha), jnp.float32),
                   jax.ShapeDtypeStruct((n, hb), jnp.float32)],
    )(x, wa, wb)


def _bias_relu_mm2_body(pa_ref, pb_ref, ba_ref, bb_ref, wa_ref, wb_ref, o_ref):
    ta = jnp.maximum(pa_ref[0] + pa_ref[1] + ba_ref[...], 0.0)
    tb = jnp.maximum(pb_ref[0] + pb_ref[1] + bb_ref[...], 0.0)
    o_ref[...] = (jnp.dot(ta, wa_ref[...], preferred_element_type=jnp.float32)
                  + jnp.dot(tb, wb_ref[...], preferred_element_type=jnp.float32))


def _bias_relu_matmul(pa, pb, ba, bb, wa, wb, blk):
    _, n, ha = pa.shape
    hb = pb.shape[2]
    cp = wa.shape[1]
    return pl.pallas_call(
        _bias_relu_mm2_body,
        grid=(n // blk,),
        in_specs=[pl.BlockSpec((2, blk, ha), lambda i: (0, i, 0)),
                  pl.BlockSpec((2, blk, hb), lambda i: (0, i, 0)),
                  pl.BlockSpec((1, ha), lambda i: (0, 0)),
                  pl.BlockSpec((1, hb), lambda i: (0, 0)),
                  pl.BlockSpec((ha, cp), lambda i: (0, 0)),
                  pl.BlockSpec((hb, cp), lambda i: (0, 0))],
        out_specs=pl.BlockSpec((blk, cp), lambda i: (i, 0)),
        out_shape=jax.ShapeDtypeStruct((n, cp), jnp.float32),
    )(pa, pb, ba, bb, wa, wb)


def _make_softmax_body(c):
    def body(p_ref, b_ref, o_ref):
        v = p_ref[0] + p_ref[1] + b_ref[...]
        col = lax.broadcasted_iota(jnp.int32, v.shape, 1)
        valid = col < c
        m = jnp.max(jnp.where(valid, v, -jnp.inf), axis=1, keepdims=True)
        e = jnp.where(valid, jnp.exp(v - m), 0.0)
        o_ref[...] = e / jnp.sum(e, axis=1, keepdims=True)
    return body


def _bias_softmax(parts, b, c, blk):
    _, n, cp = parts.shape
    return pl.pallas_call(
        _make_softmax_body(c),
        grid=(n // blk,),
        in_specs=[pl.BlockSpec((2, blk, cp), lambda i: (0, i, 0)),
                  pl.BlockSpec((1, cp), lambda i: (0, 0))],
        out_specs=pl.BlockSpec((blk, cp), lambda i: (i, 0)),
        out_shape=jax.ShapeDtypeStruct((n, cp), jnp.float32),
    )(parts, b)


def _make_agg(n, e, hp, k):
    nw = _NC * _NS
    ept = e // nw
    nb = ept // k
    rpt = n // _NS
    nslice = hp // 16
    assert e % nw == 0 and ept % k == 0 and n % _NS == 0 and hp % 16 == 0

    mesh = plsc.VectorSubcoreMesh(core_axis_name="c", subcore_axis_name="s")

    @functools.partial(
        pl.kernel,
        out_type=jax.ShapeDtypeStruct((_NC, n, hp), jnp.float32),
        mesh=mesh,
        compiler_params=pltpu.CompilerParams(use_tc_tiling_on_sc=False,
                                             needs_layout_passes=False),
        scratch_types=[
            pltpu.VMEM_SHARED((n, hp), jnp.float32),
            pltpu.VMEM((nb, k), jnp.int32),
            pltpu.VMEM((nb, k), jnp.int32),
            pltpu.VMEM((nb, k), jnp.float32),
            pltpu.VMEM((k, hp), jnp.float32),
        ],
    )
    def agg(h_hbm, src2_hbm, dst2_hbm, w2_hbm, out_hbm,
            acc, src_v, dst_v, w_v, rows_v):
        c = lax.axis_index("c")
        s = lax.axis_index("s")
        wid = c * _NS + s

        zero = jnp.zeros((16,), jnp.float32)
        for i in range(k):
            for j in range(nslice):
                rows_v[i, pl.ds(j * 16, 16)] = zero
        base = s * rpt
        full, tail = divmod(rpt, k)
        for i in range(full):
            pltpu.sync_copy(rows_v, acc.at[pl.ds(base + i * k, k)])
        if tail:
            pltpu.sync_copy(rows_v.at[pl.ds(0, tail)],
                            acc.at[pl.ds(base + full * k, tail)])

        b0 = wid * nb
        pltpu.sync_copy(src2_hbm.at[pl.ds(b0, nb)], src_v)
        pltpu.sync_copy(dst2_hbm.at[pl.ds(b0, nb)], dst_v)
        pltpu.sync_copy(w2_hbm.at[pl.ds(b0, nb)], w_v)
        plsc.subcore_barrier()

        def body(b, carry):
            pltpu.sync_copy(h_hbm.at[src_v.at[b]], rows_v)
            bvec = jnp.full((16,), b, jnp.int32)
            for i in range(k):
                wsplat = plsc.load_gather(
                    w_v, [bvec, jnp.full((16,), i, jnp.int32)])
                for j in range(nslice):
                    sl = pl.ds(j * 16, 16)
                    rows_v[i, sl] = rows_v[i, sl] * wsplat
            pltpu.sync_copy(rows_v, acc.at[dst_v.at[b]], add=True)
            return carry

        lax.fori_loop(0, nb, body, 0)
        plsc.subcore_barrier()
        pltpu.sync_copy(acc.at[pl.ds(base, rpt)],
                        out_hbm.at[c, pl.ds(base, rpt)])

    return agg


def kernel(x, edge_index, edge_weight, W1, b1, W2, b2):
    n, d = x.shape
    e = edge_index.shape[1]
    h = W1.shape[1]
    c = W2.shape[1]
    hp = ((h + 15) // 16) * 16
    cp = ((c + 15) // 16) * 16
    ha = 96
    hb = hp - ha

    src2 = edge_index[0].reshape(e // _K, _K)
    dst2 = edge_index[1].reshape(e // _K, _K)
    ew2 = edge_weight.reshape(e // _K, _K)
    w1p = jnp.pad(W1, ((0, 0), (0, hp - h)))
    b1p = jnp.pad(b1, (0, hp - h))[None, :]
    w2p = jnp.pad(W2, ((0, hp - h), (0, cp - c)))
    b2p = jnp.pad(b2, (0, cp - c))[None, :]

    h1a, h1b = _matmul_split(x, w1p[:, :ha], w1p[:, ha:], blk=1000)
    pa = _make_agg(n, e, ha, _K)(h1a, src2, dst2, ew2)
    pb = _make_agg(n, e, hb, _K)(h1b, src2, dst2, ew2)
    h2 = _bias_relu_matmul(pa, pb, b1p[:, :ha], b1p[:, ha:],
                           w2p[:ha], w2p[ha:], blk=1000)
    p2 = _make_agg(n, e, cp, _K)(h2, src2, dst2, ew2)
    out = _bias_softmax(p2, b2p, c, blk=1000)
    return out[:, :c]


# speedup vs baseline: 5.0930x; 1.4447x over previous
import functools

import jax
import jax.numpy as jnp
from jax import lax
from jax.experimental import pallas as pl
from jax.experimental.pallas import tpu as pltpu
from jax.experimental.pallas import tpu_sc as plsc

_NC = 2
_NS = 16
_K = 50


def _mm2_body(x_ref, wa_ref, wb_ref, oa_ref, ob_ref):
    x = x_ref[...]
    oa_ref[...] = jnp.dot(x, wa_ref[...], preferred_element_type=jnp.float32)
    ob_ref[...] = jnp.dot(x, wb_ref[...], preferred_element_type=jnp.float32)


def _matmul_split(x, wa, wb, blk):
    n, d = x.shape
    ha, hb = wa.shape[1], wb.shape[1]
    return pl.pallas_call(
        _mm2_body,
        grid=(n // blk,),
        in_specs=[pl.BlockSpec((blk, d), lambda i: (i, 0)),
                  pl.BlockSpec((d, ha), lambda i: (0, 0)),
                  pl.BlockSpec((d, hb), lambda i: (0, 0))],
        out_specs=[pl.BlockSpec((blk, ha), lambda i: (i, 0)),
                   pl.BlockSpec((blk, hb), lambda i: (i, 0))],
        out_shape=[jax.ShapeDtypeStruct((n, ha), jnp.float32),
                   jax.ShapeDtypeStruct((n, hb), jnp.float32)],
    )(x, wa, wb)


def _bias_relu_mm2_body(pa_ref, pb_ref, ba_ref, bb_ref, wa_ref, wb_ref, o_ref):
    ta = jnp.maximum(pa_ref[0] + pa_ref[1] + ba_ref[...], 0.0)
    tb = jnp.maximum(pb_ref[0] + pb_ref[1] + bb_ref[...], 0.0)
    o_ref[...] = (jnp.dot(ta, wa_ref[...], preferred_element_type=jnp.float32)
                  + jnp.dot(tb, wb_ref[...], preferred_element_type=jnp.float32))


def _bias_relu_matmul(pa, pb, ba, bb, wa, wb, blk):
    _, n, ha = pa.shape
    hb = pb.shape[2]
    cp = wa.shape[1]
    return pl.pallas_call(
        _bias_relu_mm2_body,
        grid=(n // blk,),
        in_specs=[pl.BlockSpec((2, blk, ha), lambda i: (0, i, 0)),
                  pl.BlockSpec((2, blk, hb), lambda i: (0, i, 0)),
                  pl.BlockSpec((1, ha), lambda i: (0, 0)),
                  pl.BlockSpec((1, hb), lambda i: (0, 0)),
                  pl.BlockSpec((ha, cp), lambda i: (0, 0)),
                  pl.BlockSpec((hb, cp), lambda i: (0, 0))],
        out_specs=pl.BlockSpec((blk, cp), lambda i: (i, 0)),
        out_shape=jax.ShapeDtypeStruct((n, cp), jnp.float32),
    )(pa, pb, ba, bb, wa, wb)


def _make_softmax_body(c):
    def body(p_ref, b_ref, o_ref):
        v = p_ref[0] + p_ref[1] + b_ref[...]
        col = lax.broadcasted_iota(jnp.int32, v.shape, 1)
        valid = col < c
        m = jnp.max(jnp.where(valid, v, -jnp.inf), axis=1, keepdims=True)
        e = jnp.where(valid, jnp.exp(v - m), 0.0)
        o_ref[...] = e / jnp.sum(e, axis=1, keepdims=True)
    return body


def _bias_softmax(parts, b, c, blk):
    _, n, cp = parts.shape
    return pl.pallas_call(
        _make_softmax_body(c),
        grid=(n // blk,),
        in_specs=[pl.BlockSpec((2, blk, cp), lambda i: (0, i, 0)),
                  pl.BlockSpec((1, cp), lambda i: (0, 0))],
        out_specs=pl.BlockSpec((blk, cp), lambda i: (i, 0)),
        out_shape=jax.ShapeDtypeStruct((n, cp), jnp.float32),
    )(parts, b)


def _make_agg(n, e, hp, k):
    nw = _NC * _NS
    ept = e // nw
    nb = ept // k
    np_ = nb // 2
    rpt = n // _NS
    nslice = hp // 16
    assert e % nw == 0 and ept % k == 0 and nb % 2 == 0
    assert n % _NS == 0 and hp % 16 == 0

    mesh = plsc.VectorSubcoreMesh(core_axis_name="c", subcore_axis_name="s")

    @functools.partial(
        pl.kernel,
        out_type=jax.ShapeDtypeStruct((_NC, n, hp), jnp.float32),
        mesh=mesh,
        compiler_params=pltpu.CompilerParams(use_tc_tiling_on_sc=False,
                                             needs_layout_passes=False),
        scratch_types=[
            pltpu.VMEM_SHARED((n, hp), jnp.float32),
            pltpu.VMEM((nb, k), jnp.int32),
            pltpu.VMEM((nb, k), jnp.int32),
            pltpu.VMEM((nb, k), jnp.float32),
            pltpu.VMEM((k, hp), jnp.float32),
            pltpu.VMEM((k, hp), jnp.float32),
            pltpu.SemaphoreType.DMA,
            pltpu.SemaphoreType.DMA,
        ],
    )
    def agg(h_hbm, src2_hbm, dst2_hbm, w2_hbm, out_hbm,
            acc, src_v, dst_v, w_v, rows0, rows1, sem0, sem1):
        c = lax.axis_index("c")
        s = lax.axis_index("s")
        wid = c * _NS + s

        zero = jnp.zeros((16,), jnp.float32)
        for i in range(k):
            for j in range(nslice):
                rows0[i, pl.ds(j * 16, 16)] = zero
        base = s * rpt
        full, tail = divmod(rpt, k)
        for i in range(full):
            pltpu.sync_copy(rows0, acc.at[pl.ds(base + i * k, k)])
        if tail:
            pltpu.sync_copy(rows0.at[pl.ds(0, tail)],
                            acc.at[pl.ds(base + full * k, tail)])

        b0 = wid * nb
        pltpu.sync_copy(src2_hbm.at[pl.ds(b0, nb)], src_v)
        pltpu.sync_copy(dst2_hbm.at[pl.ds(b0, nb)], dst_v)
        pltpu.sync_copy(w2_hbm.at[pl.ds(b0, nb)], w_v)
        plsc.subcore_barrier()

        def scale(rows, b):
            bvec = jnp.full((16,), b, jnp.int32)
            for i in range(k):
                wsplat = plsc.load_gather(
                    w_v, [bvec, jnp.full((16,), i, jnp.int32)])
                for j in range(nslice):
                    sl = pl.ds(j * 16, 16)
                    rows[i, sl] = rows[i, sl] * wsplat

        pltpu.async_copy(h_hbm.at[src_v.at[0]], rows0, sem0)

        def body(p, carry):
            a = 2 * p
            b = a + 1
            pltpu.make_async_copy(h_hbm.at[src_v.at[a]], rows0, sem0).wait()
            pltpu.async_copy(h_hbm.at[src_v.at[b]], rows1, sem1)
            scale(rows0, a)
            pltpu.sync_copy(rows0, acc.at[dst_v.at[a]], add=True)
            pltpu.make_async_copy(h_hbm.at[src_v.at[b]], rows1, sem1).wait()

            @pl.when(p + 1 < np_)
            def _():
                pltpu.async_copy(h_hbm.at[src_v.at[a + 2]], rows0, sem0)

            scale(rows1, b)
            pltpu.sync_copy(rows1, acc.at[dst_v.at[b]], add=True)
            return carry

        lax.fori_loop(0, np_, body, 0)
        plsc.subcore_barrier()
        pltpu.sync_copy(acc.at[pl.ds(base, rpt)],
                        out_hbm.at[c, pl.ds(base, rpt)])

    return agg


def kernel(x, edge_index, edge_weight, W1, b1, W2, b2):
    n, d = x.shape
    e = edge_index.shape[1]
    h = W1.shape[1]
    c = W2.shape[1]
    hp = ((h + 15) // 16) * 16
    cp = ((c + 15) // 16) * 16
    ha = 96
    hb = hp - ha

    src2 = edge_index[0].reshape(e // _K, _K)
    dst2 = edge_index[1].reshape(e // _K, _K)
    ew2 = edge_weight.reshape(e // _K, _K)
    w1p = jnp.pad(W1, ((0, 0), (0, hp - h)))
    b1p = jnp.pad(b1, (0, hp - h))[None, :]
    w2p = jnp.pad(W2, ((0, hp - h), (0, cp - c)))
    b2p = jnp.pad(b2, (0, cp - c))[None, :]

    h1a, h1b = _matmul_split(x, w1p[:, :ha], w1p[:, ha:], blk=1000)
    pa = _make_agg(n, e, ha, _K)(h1a, src2, dst2, ew2)
    pb = _make_agg(n, e, hb, _K)(h1b, src2, dst2, ew2)
    h2 = _bias_relu_matmul(pa, pb, b1p[:, :ha], b1p[:, ha:],
                           w2p[:ha], w2p[ha:], blk=1000)
    p2 = _make_agg(n, e, cp, _K)(h2, src2, dst2, ew2)
    out = _bias_softmax(p2, b2p, c, blk=1000)
    return out[:, :c]
